# baseline (speedup 1.0000x reference)
import functools

import jax
import jax.numpy as jnp
from jax import lax
from jax.experimental import pallas as pl
from jax.experimental.pallas import tpu as pltpu
from jax.experimental.pallas import tpu_sc as plsc

NUM_EXPERTS = 8
TOP_K = 2
HIDDEN = 1024
INTER = 2048
TOKENS = 2048

RCHUNK = 256
NRC = TOKENS // RCHUNK
TILE_M = 256
NT = (TOKENS * TOP_K) // TILE_M + NUM_EXPERTS
M_PAD = NT * TILE_M

NC, NS, LANES = 2, 16, 16
NW = NC * NS


def _router_body(logits_ref, i0_ref, i1_ref, r0_ref, r1_ref, w0_ref, w1_ref,
                 cnt_ref, carry_ref):
    tc = pl.program_id(0)
    logits = logits_ref[...].astype(jnp.float32)
    m = jnp.max(logits, axis=1, keepdims=True)
    p = jnp.exp(logits - m)
    p = p / jnp.sum(p, axis=1, keepdims=True)
    eiota = lax.broadcasted_iota(jnp.int32, p.shape, 1)
    w0 = jnp.max(p, axis=1)
    i0 = jnp.min(jnp.where(p == w0[:, None], eiota, NUM_EXPERTS), axis=1)
    p1 = jnp.where(eiota == i0[:, None], -1.0, p)
    w1 = jnp.max(p1, axis=1)
    i1 = jnp.min(jnp.where(p1 == w1[:, None], eiota, NUM_EXPERTS), axis=1)
    denom = w0 + w1

    lane = lax.broadcasted_iota(jnp.int32, (RCHUNK, 128), 1)
    oh0 = (lane == i0[:, None]).astype(jnp.float32)
    oh1 = (lane == i1[:, None]).astype(jnp.float32)
    oh = (oh0 + oh1).astype(jnp.bfloat16)
    r = lax.broadcasted_iota(jnp.int32, (RCHUNK, RCHUNK), 0)
    c = lax.broadcasted_iota(jnp.int32, (RCHUNK, RCHUNK), 1)
    tril = (r > c).astype(jnp.bfloat16)
    cexcl = lax.dot_general(tril, oh, (((1,), (0,)), ((), ())),
                            preferred_element_type=jnp.float32)

    @pl.when(tc == 0)
    def _():
        carry_ref[...] = jnp.zeros_like(carry_ref)

    carry = carry_ref[...]
    cexcl = cexcl + carry
    r0 = jnp.sum(cexcl * oh0, axis=1)
    r1 = jnp.sum(cexcl * oh1, axis=1)
    new_carry = carry + jnp.sum(oh0 + oh1, axis=0, keepdims=True)
    carry_ref[...] = new_carry

    i0_ref[...] = i0.reshape(1, 1, RCHUNK)
    i1_ref[...] = i1.reshape(1, 1, RCHUNK)
    r0_ref[...] = r0.astype(jnp.int32).reshape(1, 1, RCHUNK)
    r1_ref[...] = r1.astype(jnp.int32).reshape(1, 1, RCHUNK)
    w0_ref[...] = (w0 / denom).reshape(1, 1, RCHUNK)
    w1_ref[...] = (w1 / denom).reshape(1, 1, RCHUNK)
    cnt_ref[...] = new_carry.astype(jnp.int32)


def _router(router_logits):
    outs = pl.pallas_call(
        _router_body,
        grid=(NRC,),
        in_specs=[pl.BlockSpec((RCHUNK, NUM_EXPERTS), lambda tc: (tc, 0))],
        out_specs=[
            pl.BlockSpec((1, 1, RCHUNK), lambda tc: (tc, 0, 0)),
            pl.BlockSpec((1, 1, RCHUNK), lambda tc: (tc, 0, 0)),
            pl.BlockSpec((1, 1, RCHUNK), lambda tc: (tc, 0, 0)),
            pl.BlockSpec((1, 1, RCHUNK), lambda tc: (tc, 0, 0)),
            pl.BlockSpec((1, 1, RCHUNK), lambda tc: (tc, 0, 0)),
            pl.BlockSpec((1, 1, RCHUNK), lambda tc: (tc, 0, 0)),
            pl.BlockSpec((1, 128), lambda tc: (0, 0)),
        ],
        out_shape=[
            jax.ShapeDtypeStruct((NRC, 1, RCHUNK), jnp.int32),
            jax.ShapeDtypeStruct((NRC, 1, RCHUNK), jnp.int32),
            jax.ShapeDtypeStruct((NRC, 1, RCHUNK), jnp.int32),
            jax.ShapeDtypeStruct((NRC, 1, RCHUNK), jnp.int32),
            jax.ShapeDtypeStruct((NRC, 1, RCHUNK), jnp.float32),
            jax.ShapeDtypeStruct((NRC, 1, RCHUNK), jnp.float32),
            jax.ShapeDtypeStruct((1, 128), jnp.int32),
        ],
        scratch_shapes=[pltpu.VMEM((1, 128), jnp.float32)],
        compiler_params=pltpu.CompilerParams(
            dimension_semantics=("arbitrary",)),
    )(router_logits)
    return outs


@functools.lru_cache(maxsize=1)
def _sc_mesh():
    return plsc.VectorSubcoreMesh(core_axis_name="c", subcore_axis_name="s",
                                  num_cores=NC, num_subcores=NS)


def _meta_body(cnt_hbm, i0_hbm, i1_hbm, r0_hbm, r1_hbm, w0_hbm, w1_hbm,
               src_hbm, wsort_hbm, pos0_hbm, pos1_hbm, te_hbm,
               cnt_v, starts_v, pc_v, i0_v, i1_v, r0_v, r1_v, w0_v, w1_v,
               src_v, wsort_v, pos0_v, pos1_v, te_v):
    cid = lax.axis_index("c")
    sid = lax.axis_index("s")

    @pl.when(jnp.logical_and(cid == 0, sid == 0))
    def _():
        pltpu.sync_copy(cnt_hbm.at[pl.ds(0, 16)], cnt_v)
        pltpu.sync_copy(i0_hbm, i0_v)
        pltpu.sync_copy(i1_hbm, i1_v)
        pltpu.sync_copy(r0_hbm, r0_v)
        pltpu.sync_copy(r1_hbm, r1_v)
        pltpu.sync_copy(w0_hbm, w0_v)
        pltpu.sync_copy(w1_hbm, w1_v)

        lane = lax.broadcasted_iota(jnp.int32, (LANES,), 0)
        c = cnt_v[...]
        pc = ((c + (TILE_M - 1)) >> 8) << 8
        pc = jnp.where(lane < NUM_EXPERTS, pc, 0)
        pc_v[...] = pc
        starts = jnp.zeros((LANES,), jnp.int32)
        for k in range(1, NUM_EXPERTS):
            g = plsc.load_gather(pc_v, [jnp.maximum(lane - k, 0)])
            starts = starts + jnp.where(lane >= k, g, 0)
        starts_v[...] = starts

        zeros16 = jnp.zeros((LANES,), jnp.int32)
        m0 = lane * TILE_M
        m1 = (lane + LANES) * TILE_M
        te0 = zeros16
        te1 = zeros16
        for e in range(1, NUM_EXPERTS):
            st_b = plsc.load_gather(starts_v,
                                    [jnp.full((LANES,), e, jnp.int32)])
            te0 = te0 + (m0 >= st_b).astype(jnp.int32)
            te1 = te1 + (m1 >= st_b).astype(jnp.int32)
        te_v[pl.ds(0, 16)] = jnp.clip(te0, 0, NUM_EXPERTS - 1)
        te_v[pl.ds(16, 16)] = jnp.clip(te1, 0, NUM_EXPERTS - 1)
        pltpu.sync_copy(te_v, te_hbm)

        def zero_body(i, _):
            src_v[pl.ds(i * LANES, LANES)] = zeros16
            wsort_v[pl.ds(i * LANES, LANES)] = jnp.zeros((LANES,), jnp.float32)
            return _

        lax.fori_loop(0, M_PAD // LANES, zero_body, None)

        def assign_body(i, _):
            base = i * LANES
            toks = base + lane
            e0 = i0_v[pl.ds(base, LANES)]
            p0 = plsc.load_gather(starts_v, [e0]) + r0_v[pl.ds(base, LANES)]
            pos0_v[pl.ds(base, LANES)] = p0
            plsc.store_scatter(src_v, [p0], toks)
            plsc.store_scatter(wsort_v, [p0], w0_v[pl.ds(base, LANES)])
            e1 = i1_v[pl.ds(base, LANES)]
            p1 = plsc.load_gather(starts_v, [e1]) + r1_v[pl.ds(base, LANES)]
            pos1_v[pl.ds(base, LANES)] = p1
            plsc.store_scatter(src_v, [p1], toks)
            plsc.store_scatter(wsort_v, [p1], w1_v[pl.ds(base, LANES)])
            return _

        lax.fori_loop(0, TOKENS // LANES, assign_body, None)

        pltpu.sync_copy(src_v, src_hbm)
        pltpu.sync_copy(wsort_v, wsort_hbm)
        pltpu.sync_copy(pos0_v, pos0_hbm)
        pltpu.sync_copy(pos1_v, pos1_hbm)


def _metadata(cnt, i0, i1, r0, r1, w0, w1):
    f = pl.kernel(
        _meta_body,
        out_type=[
            jax.ShapeDtypeStruct((M_PAD,), jnp.int32),
            jax.ShapeDtypeStruct((M_PAD,), jnp.float32),
            jax.ShapeDtypeStruct((TOKENS,), jnp.int32),
            jax.ShapeDtypeStruct((TOKENS,), jnp.int32),
            jax.ShapeDtypeStruct((NW,), jnp.int32),
        ],
        mesh=_sc_mesh(),
        compiler_params=pltpu.CompilerParams(needs_layout_passes=False),
        scratch_types=[
            pltpu.VMEM((LANES,), jnp.int32),
            pltpu.VMEM((LANES,), jnp.int32),
            pltpu.VMEM((LANES,), jnp.int32),
            pltpu.VMEM((TOKENS,), jnp.int32),
            pltpu.VMEM((TOKENS,), jnp.int32),
            pltpu.VMEM((TOKENS,), jnp.int32),
            pltpu.VMEM((TOKENS,), jnp.int32),
            pltpu.VMEM((TOKENS,), jnp.float32),
            pltpu.VMEM((TOKENS,), jnp.float32),
            pltpu.VMEM((M_PAD,), jnp.int32),
            pltpu.VMEM((M_PAD,), jnp.float32),
            pltpu.VMEM((TOKENS,), jnp.int32),
            pltpu.VMEM((TOKENS,), jnp.int32),
            pltpu.VMEM((NW,), jnp.int32),
        ],
    )
    return f(cnt, i0, i1, r0, r1, w0, w1)


G_ROWS = M_PAD // NW
G_CHUNK = 96


def _gather_body(src_hbm, x_hbm, xs_hbm, idx_v, buf_v, sem):
    wid = lax.axis_index("s") * NC + lax.axis_index("c")
    base = wid * G_ROWS

    def chunk(j, _):
        b = base + j * G_CHUNK
        pltpu.sync_copy(src_hbm.at[pl.ds(b, G_CHUNK)], idx_v)
        pltpu.async_copy(x_hbm.at[idx_v], buf_v, sem).wait()
        pltpu.sync_copy(buf_v, xs_hbm.at[pl.ds(b, G_CHUNK)])
        return _

    lax.fori_loop(0, G_ROWS // G_CHUNK, chunk, None)


def _gather(src, hidden_states):
    f = pl.kernel(
        _gather_body,
        out_type=[jax.ShapeDtypeStruct((M_PAD, HIDDEN), jnp.float32)],
        mesh=_sc_mesh(),
        compiler_params=pltpu.CompilerParams(needs_layout_passes=False),
        scratch_types=[
            pltpu.VMEM((G_CHUNK,), jnp.int32),
            pltpu.VMEM((G_CHUNK, HIDDEN), jnp.float32),
            pltpu.SemaphoreType.DMA,
        ],
    )
    return f(src, hidden_states)[0]


def _mm_body(te_ref, xs_ref, w13_ref, w2_ref, ws_ref, o_ref):
    x = xs_ref[...].astype(jnp.bfloat16)
    w13 = w13_ref[0]
    w2 = w2_ref[0]
    h = lax.dot_general(x, w13, (((1,), (1,)), ((), ())),
                        preferred_element_type=jnp.float32)
    gate = h[:, :INTER]
    up = h[:, INTER:]
    act = (gate * jax.nn.sigmoid(gate) * up).astype(jnp.bfloat16)
    y = lax.dot_general(act, w2, (((1,), (1,)), ((), ())),
                        preferred_element_type=jnp.float32)
    o_ref[...] = y * ws_ref[0, 0, :][:, None]


def _grouped_matmul(te, xs, w13_bf, w2_bf, wsort):
    grid_spec = pltpu.PrefetchScalarGridSpec(
        num_scalar_prefetch=1,
        grid=(NT,),
        in_specs=[
            pl.BlockSpec((TILE_M, HIDDEN), lambda j, te: (j, 0)),
            pl.BlockSpec((1, 2 * INTER, HIDDEN), lambda j, te: (te[j], 0, 0)),
            pl.BlockSpec((1, HIDDEN, INTER), lambda j, te: (te[j], 0, 0)),
            pl.BlockSpec((1, 1, TILE_M), lambda j, te: (j, 0, 0)),
        ],
        out_specs=pl.BlockSpec((TILE_M, HIDDEN), lambda j, te: (j, 0)),
    )
    return pl.pallas_call(
        _mm_body,
        grid_spec=grid_spec,
        out_shape=jax.ShapeDtypeStruct((M_PAD, HIDDEN), jnp.float32),
        compiler_params=pltpu.CompilerParams(
            dimension_semantics=("arbitrary",)),
    )(te, xs, w13_bf, w2_bf, wsort.reshape(NT, 1, TILE_M))


C_TOKS = TOKENS // NW
C_CHUNK = 32


def _combine_body(pos0_hbm, pos1_hbm, ys_hbm, out_hbm,
                  p0_v, p1_v, buf0_v, buf1_v, sem0, sem1):
    wid = lax.axis_index("s") * NC + lax.axis_index("c")
    base = wid * C_TOKS

    def chunk(j, _):
        b = base + j * C_CHUNK
        pltpu.sync_copy(pos0_hbm.at[pl.ds(b, C_CHUNK)], p0_v)
        pltpu.sync_copy(pos1_hbm.at[pl.ds(b, C_CHUNK)], p1_v)
        cp0 = pltpu.async_copy(ys_hbm.at[p0_v], buf0_v, sem0)
        cp1 = pltpu.async_copy(ys_hbm.at[p1_v], buf1_v, sem1)
        cp0.wait()
        cp1.wait()

        def add_body(it, _):
            row = it // (HIDDEN // LANES)
            off = (it % (HIDDEN // LANES)) * LANES
            buf0_v[row, pl.ds(off, LANES)] = (buf0_v[row, pl.ds(off, LANES)]
                                              + buf1_v[row, pl.ds(off, LANES)])
            return _

        lax.fori_loop(0, (C_CHUNK * HIDDEN) // LANES, add_body, None)
        pltpu.sync_copy(buf0_v, out_hbm.at[pl.ds(b, C_CHUNK)])
        return _

    lax.fori_loop(0, C_TOKS // C_CHUNK, chunk, None)


def _combine(pos0, pos1, ysort):
    f = pl.kernel(
        _combine_body,
        out_type=[jax.ShapeDtypeStruct((TOKENS, HIDDEN), jnp.float32)],
        mesh=_sc_mesh(),
        compiler_params=pltpu.CompilerParams(needs_layout_passes=False),
        scratch_types=[
            pltpu.VMEM((C_CHUNK,), jnp.int32),
            pltpu.VMEM((C_CHUNK,), jnp.int32),
            pltpu.VMEM((C_CHUNK, HIDDEN), jnp.float32),
            pltpu.VMEM((C_CHUNK, HIDDEN), jnp.float32),
            pltpu.SemaphoreType.DMA,
            pltpu.SemaphoreType.DMA,
        ],
    )
    return f(pos0, pos1, ysort)[0]


@jax.jit
def kernel(hidden_states, router_logits, w13_weight, w2_weight):
    i0, i1, r0, r1, w0, w1, cnt = _router(router_logits)
    src, wsort, pos0, pos1, te = _metadata(
        cnt.reshape(128),
        i0.reshape(TOKENS), i1.reshape(TOKENS),
        r0.reshape(TOKENS), r1.reshape(TOKENS),
        w0.reshape(TOKENS), w1.reshape(TOKENS))
    xs = _gather(src, hidden_states)
    w13_bf = w13_weight.astype(jnp.bfloat16)
    w2_bf = w2_weight.astype(jnp.bfloat16)
    ysort = _grouped_matmul(te, xs, w13_bf, w2_bf, wsort)
    return _combine(pos0, pos1, ysort)


# speedup vs baseline: 1.4025x; 1.4025x over previous
import functools

import jax
import jax.numpy as jnp
from jax import lax
from jax.experimental import pallas as pl
from jax.experimental.pallas import tpu as pltpu
from jax.experimental.pallas import tpu_sc as plsc

NUM_EXPERTS = 8
TOP_K = 2
HIDDEN = 1024
INTER = 2048
TOKENS = 2048

RCHUNK = 256
NRC = TOKENS // RCHUNK
TILE_M = 256
NT = (TOKENS * TOP_K) // TILE_M + NUM_EXPERTS
M_PAD = NT * TILE_M

NC, NS, LANES = 2, 16, 16
NW = NC * NS


def _router_body(logits_ref, i0_ref, i1_ref, r0_ref, r1_ref, w0_ref, w1_ref,
                 cnt_ref, carry_ref):
    tc = pl.program_id(0)
    logits = logits_ref[...].astype(jnp.float32)
    m = jnp.max(logits, axis=1, keepdims=True)
    p = jnp.exp(logits - m)
    p = p / jnp.sum(p, axis=1, keepdims=True)
    eiota = lax.broadcasted_iota(jnp.int32, p.shape, 1)
    w0 = jnp.max(p, axis=1)
    i0 = jnp.min(jnp.where(p == w0[:, None], eiota, NUM_EXPERTS), axis=1)
    p1 = jnp.where(eiota == i0[:, None], -1.0, p)
    w1 = jnp.max(p1, axis=1)
    i1 = jnp.min(jnp.where(p1 == w1[:, None], eiota, NUM_EXPERTS), axis=1)
    denom = w0 + w1

    lane = lax.broadcasted_iota(jnp.int32, (RCHUNK, 128), 1)
    oh0 = (lane == i0[:, None]).astype(jnp.float32)
    oh1 = (lane == i1[:, None]).astype(jnp.float32)
    oh = (oh0 + oh1).astype(jnp.bfloat16)
    r = lax.broadcasted_iota(jnp.int32, (RCHUNK, RCHUNK), 0)
    c = lax.broadcasted_iota(jnp.int32, (RCHUNK, RCHUNK), 1)
    tril = (r > c).astype(jnp.bfloat16)
    cexcl = lax.dot_general(tril, oh, (((1,), (0,)), ((), ())),
                            preferred_element_type=jnp.float32)

    @pl.when(tc == 0)
    def _():
        carry_ref[...] = jnp.zeros_like(carry_ref)

    carry = carry_ref[...]
    cexcl = cexcl + carry
    r0 = jnp.sum(cexcl * oh0, axis=1)
    r1 = jnp.sum(cexcl * oh1, axis=1)
    new_carry = carry + jnp.sum(oh0 + oh1, axis=0, keepdims=True)
    carry_ref[...] = new_carry

    i0_ref[...] = i0.reshape(1, 1, RCHUNK)
    i1_ref[...] = i1.reshape(1, 1, RCHUNK)
    r0_ref[...] = r0.astype(jnp.int32).reshape(1, 1, RCHUNK)
    r1_ref[...] = r1.astype(jnp.int32).reshape(1, 1, RCHUNK)
    w0_ref[...] = (w0 / denom).reshape(1, 1, RCHUNK)
    w1_ref[...] = (w1 / denom).reshape(1, 1, RCHUNK)
    cnt_ref[...] = new_carry.astype(jnp.int32)


def _router(router_logits):
    outs = pl.pallas_call(
        _router_body,
        grid=(NRC,),
        in_specs=[pl.BlockSpec((RCHUNK, NUM_EXPERTS), lambda tc: (tc, 0))],
        out_specs=[
            pl.BlockSpec((1, 1, RCHUNK), lambda tc: (tc, 0, 0)),
            pl.BlockSpec((1, 1, RCHUNK), lambda tc: (tc, 0, 0)),
            pl.BlockSpec((1, 1, RCHUNK), lambda tc: (tc, 0, 0)),
            pl.BlockSpec((1, 1, RCHUNK), lambda tc: (tc, 0, 0)),
            pl.BlockSpec((1, 1, RCHUNK), lambda tc: (tc, 0, 0)),
            pl.BlockSpec((1, 1, RCHUNK), lambda tc: (tc, 0, 0)),
            pl.BlockSpec((1, 128), lambda tc: (0, 0)),
        ],
        out_shape=[
            jax.ShapeDtypeStruct((NRC, 1, RCHUNK), jnp.int32),
            jax.ShapeDtypeStruct((NRC, 1, RCHUNK), jnp.int32),
            jax.ShapeDtypeStruct((NRC, 1, RCHUNK), jnp.int32),
            jax.ShapeDtypeStruct((NRC, 1, RCHUNK), jnp.int32),
            jax.ShapeDtypeStruct((NRC, 1, RCHUNK), jnp.float32),
            jax.ShapeDtypeStruct((NRC, 1, RCHUNK), jnp.float32),
            jax.ShapeDtypeStruct((1, 128), jnp.int32),
        ],
        scratch_shapes=[pltpu.VMEM((1, 128), jnp.float32)],
        compiler_params=pltpu.CompilerParams(
            dimension_semantics=("arbitrary",)),
    )(router_logits)
    return outs


@functools.lru_cache(maxsize=1)
def _sc_mesh():
    return plsc.VectorSubcoreMesh(core_axis_name="c", subcore_axis_name="s",
                                  num_cores=NC, num_subcores=NS)


def _meta_body(cnt_hbm, i0_hbm, i1_hbm, r0_hbm, r1_hbm, w0_hbm, w1_hbm,
               src_hbm, wsort_hbm, pos0_hbm, pos1_hbm, te_hbm,
               cnt_v, starts_v, pc_v, i0_v, i1_v, r0_v, r1_v, w0_v, w1_v,
               src_v, wsort_v, pos0_v, pos1_v, te_v):
    cid = lax.axis_index("c")
    sid = lax.axis_index("s")

    @pl.when(jnp.logical_and(cid == 0, sid == 0))
    def _():
        pltpu.sync_copy(cnt_hbm.at[pl.ds(0, 16)], cnt_v)
        pltpu.sync_copy(i0_hbm, i0_v)
        pltpu.sync_copy(i1_hbm, i1_v)
        pltpu.sync_copy(r0_hbm, r0_v)
        pltpu.sync_copy(r1_hbm, r1_v)
        pltpu.sync_copy(w0_hbm, w0_v)
        pltpu.sync_copy(w1_hbm, w1_v)

        lane = lax.broadcasted_iota(jnp.int32, (LANES,), 0)
        c = cnt_v[...]
        pc = ((c + (TILE_M - 1)) >> 8) << 8
        pc = jnp.where(lane < NUM_EXPERTS, pc, 0)
        pc_v[...] = pc
        starts = jnp.zeros((LANES,), jnp.int32)
        for k in range(1, NUM_EXPERTS):
            g = plsc.load_gather(pc_v, [jnp.maximum(lane - k, 0)])
            starts = starts + jnp.where(lane >= k, g, 0)
        starts_v[...] = starts

        zeros16 = jnp.zeros((LANES,), jnp.int32)
        m0 = lane * TILE_M
        m1 = (lane + LANES) * TILE_M
        te0 = zeros16
        te1 = zeros16
        for e in range(1, NUM_EXPERTS):
            st_b = plsc.load_gather(starts_v,
                                    [jnp.full((LANES,), e, jnp.int32)])
            te0 = te0 + (m0 >= st_b).astype(jnp.int32)
            te1 = te1 + (m1 >= st_b).astype(jnp.int32)
        te_v[pl.ds(0, 16)] = jnp.clip(te0, 0, NUM_EXPERTS - 1)
        te_v[pl.ds(16, 16)] = jnp.clip(te1, 0, NUM_EXPERTS - 1)
        pltpu.sync_copy(te_v, te_hbm)

        def zero_body(i, _):
            src_v[pl.ds(i * LANES, LANES)] = zeros16
            wsort_v[pl.ds(i * LANES, LANES)] = jnp.zeros((LANES,), jnp.float32)
            return _

        lax.fori_loop(0, M_PAD // LANES, zero_body, None)

        def assign_body(i, _):
            base = i * LANES
            toks = base + lane
            e0 = i0_v[pl.ds(base, LANES)]
            p0 = plsc.load_gather(starts_v, [e0]) + r0_v[pl.ds(base, LANES)]
            pos0_v[pl.ds(base, LANES)] = p0
            plsc.store_scatter(src_v, [p0], toks)
            plsc.store_scatter(wsort_v, [p0], w0_v[pl.ds(base, LANES)])
            e1 = i1_v[pl.ds(base, LANES)]
            p1 = plsc.load_gather(starts_v, [e1]) + r1_v[pl.ds(base, LANES)]
            pos1_v[pl.ds(base, LANES)] = p1
            plsc.store_scatter(src_v, [p1], toks)
            plsc.store_scatter(wsort_v, [p1], w1_v[pl.ds(base, LANES)])
            return _

        lax.fori_loop(0, TOKENS // LANES, assign_body, None)

        pltpu.sync_copy(src_v, src_hbm)
        pltpu.sync_copy(wsort_v, wsort_hbm)
        pltpu.sync_copy(pos0_v, pos0_hbm)
        pltpu.sync_copy(pos1_v, pos1_hbm)


def _metadata(cnt, i0, i1, r0, r1, w0, w1):
    f = pl.kernel(
        _meta_body,
        out_type=[
            jax.ShapeDtypeStruct((M_PAD,), jnp.int32),
            jax.ShapeDtypeStruct((M_PAD,), jnp.float32),
            jax.ShapeDtypeStruct((TOKENS,), jnp.int32),
            jax.ShapeDtypeStruct((TOKENS,), jnp.int32),
            jax.ShapeDtypeStruct((NW,), jnp.int32),
        ],
        mesh=_sc_mesh(),
        compiler_params=pltpu.CompilerParams(needs_layout_passes=False),
        scratch_types=[
            pltpu.VMEM((LANES,), jnp.int32),
            pltpu.VMEM((LANES,), jnp.int32),
            pltpu.VMEM((LANES,), jnp.int32),
            pltpu.VMEM((TOKENS,), jnp.int32),
            pltpu.VMEM((TOKENS,), jnp.int32),
            pltpu.VMEM((TOKENS,), jnp.int32),
            pltpu.VMEM((TOKENS,), jnp.int32),
            pltpu.VMEM((TOKENS,), jnp.float32),
            pltpu.VMEM((TOKENS,), jnp.float32),
            pltpu.VMEM((M_PAD,), jnp.int32),
            pltpu.VMEM((M_PAD,), jnp.float32),
            pltpu.VMEM((TOKENS,), jnp.int32),
            pltpu.VMEM((TOKENS,), jnp.int32),
            pltpu.VMEM((NW,), jnp.int32),
        ],
    )
    return f(cnt, i0, i1, r0, r1, w0, w1)


G_ROWS = M_PAD // NW
G_CHUNK = 96


def _gather_body(src_hbm, x_hbm, xs_hbm, idx_v, buf_v, sem):
    wid = lax.axis_index("s") * NC + lax.axis_index("c")
    base = wid * G_ROWS

    def chunk(j, _):
        b = base + j * G_CHUNK
        pltpu.sync_copy(src_hbm.at[pl.ds(b, G_CHUNK)], idx_v)
        pltpu.async_copy(x_hbm.at[idx_v], buf_v, sem).wait()
        pltpu.sync_copy(buf_v, xs_hbm.at[pl.ds(b, G_CHUNK)])
        return _

    lax.fori_loop(0, G_ROWS // G_CHUNK, chunk, None)


def _gather(src, hidden_states):
    f = pl.kernel(
        _gather_body,
        out_type=[jax.ShapeDtypeStruct((M_PAD, HIDDEN), jnp.float32)],
        mesh=_sc_mesh(),
        compiler_params=pltpu.CompilerParams(needs_layout_passes=False),
        scratch_types=[
            pltpu.VMEM((G_CHUNK,), jnp.int32),
            pltpu.VMEM((G_CHUNK, HIDDEN), jnp.float32),
            pltpu.SemaphoreType.DMA,
        ],
    )
    return f(src, hidden_states)[0]


def _mm_body(te_ref, src_ref, x_ref, w13_ref, w2_ref, ws_ref, o_ref):
    s = src_ref[0, 0, :]
    tok = lax.broadcasted_iota(jnp.int32, (TILE_M, TOKENS), 1)
    perm = (tok == s[:, None]).astype(jnp.bfloat16)
    xt = lax.dot_general(perm, x_ref[...], (((1,), (0,)), ((), ())),
                         preferred_element_type=jnp.float32)
    x = xt.astype(jnp.bfloat16)
    w13 = w13_ref[0]
    w2 = w2_ref[0]
    h = lax.dot_general(x, w13, (((1,), (1,)), ((), ())),
                        preferred_element_type=jnp.float32)
    gate = h[:, :INTER]
    up = h[:, INTER:]
    act = (gate * jax.nn.sigmoid(gate) * up).astype(jnp.bfloat16)
    y = lax.dot_general(act, w2, (((1,), (1,)), ((), ())),
                        preferred_element_type=jnp.float32)
    o_ref[...] = y * ws_ref[0, 0, :][:, None]


def _grouped_matmul(te, src, x_bf, w13_bf, w2_bf, wsort):
    grid_spec = pltpu.PrefetchScalarGridSpec(
        num_scalar_prefetch=1,
        grid=(NT,),
        in_specs=[
            pl.BlockSpec((1, 1, TILE_M), lambda j, te: (j, 0, 0)),
            pl.BlockSpec((TOKENS, HIDDEN), lambda j, te: (0, 0)),
            pl.BlockSpec((1, 2 * INTER, HIDDEN), lambda j, te: (te[j], 0, 0)),
            pl.BlockSpec((1, HIDDEN, INTER), lambda j, te: (te[j], 0, 0)),
            pl.BlockSpec((1, 1, TILE_M), lambda j, te: (j, 0, 0)),
        ],
        out_specs=pl.BlockSpec((TILE_M, HIDDEN), lambda j, te: (j, 0)),
    )
    return pl.pallas_call(
        _mm_body,
        grid_spec=grid_spec,
        out_shape=jax.ShapeDtypeStruct((M_PAD, HIDDEN), jnp.float32),
        compiler_params=pltpu.CompilerParams(
            dimension_semantics=("arbitrary",)),
    )(te, src.reshape(NT, 1, TILE_M), x_bf, w13_bf, w2_bf,
      wsort.reshape(NT, 1, TILE_M))


C_TOKS = TOKENS // NW
C_CHUNK = 32


def _combine_body(pos0_hbm, pos1_hbm, ys_hbm, out_hbm,
                  p0_v, p1_v, buf0_v, buf1_v, sem0, sem1):
    wid = lax.axis_index("s") * NC + lax.axis_index("c")
    base = wid * C_TOKS

    def chunk(j, _):
        b = base + j * C_CHUNK
        pltpu.sync_copy(pos0_hbm.at[pl.ds(b, C_CHUNK)], p0_v)
        pltpu.sync_copy(pos1_hbm.at[pl.ds(b, C_CHUNK)], p1_v)
        cp0 = pltpu.async_copy(ys_hbm.at[p0_v], buf0_v, sem0)
        cp1 = pltpu.async_copy(ys_hbm.at[p1_v], buf1_v, sem1)
        cp0.wait()
        cp1.wait()

        def add_body(it, _):
            row = it // (HIDDEN // LANES)
            off = (it % (HIDDEN // LANES)) * LANES
            buf0_v[row, pl.ds(off, LANES)] = (buf0_v[row, pl.ds(off, LANES)]
                                              + buf1_v[row, pl.ds(off, LANES)])
            return _

        lax.fori_loop(0, (C_CHUNK * HIDDEN) // LANES, add_body, None)
        pltpu.sync_copy(buf0_v, out_hbm.at[pl.ds(b, C_CHUNK)])
        return _

    lax.fori_loop(0, C_TOKS // C_CHUNK, chunk, None)


def _combine(pos0, pos1, ysort):
    f = pl.kernel(
        _combine_body,
        out_type=[jax.ShapeDtypeStruct((TOKENS, HIDDEN), jnp.float32)],
        mesh=_sc_mesh(),
        compiler_params=pltpu.CompilerParams(needs_layout_passes=False),
        scratch_types=[
            pltpu.VMEM((C_CHUNK,), jnp.int32),
            pltpu.VMEM((C_CHUNK,), jnp.int32),
            pltpu.VMEM((C_CHUNK, HIDDEN), jnp.float32),
            pltpu.VMEM((C_CHUNK, HIDDEN), jnp.float32),
            pltpu.SemaphoreType.DMA,
            pltpu.SemaphoreType.DMA,
        ],
    )
    return f(pos0, pos1, ysort)[0]


@jax.jit
def kernel(hidden_states, router_logits, w13_weight, w2_weight):
    i0, i1, r0, r1, w0, w1, cnt = _router(router_logits)
    src, wsort, pos0, pos1, te = _metadata(
        cnt.reshape(128),
        i0.reshape(TOKENS), i1.reshape(TOKENS),
        r0.reshape(TOKENS), r1.reshape(TOKENS),
        w0.reshape(TOKENS), w1.reshape(TOKENS))
    x_bf = hidden_states.astype(jnp.bfloat16)
    w13_bf = w13_weight.astype(jnp.bfloat16)
    w2_bf = w2_weight.astype(jnp.bfloat16)
    ysort = _grouped_matmul(te, src, x_bf, w13_bf, w2_bf, wsort)
    return _combine(pos0, pos1, ysort)


# speedup vs baseline: 1.7152x; 1.2230x over previous
import functools

import jax
import jax.numpy as jnp
from jax import lax
from jax.experimental import pallas as pl
from jax.experimental.pallas import tpu as pltpu
from jax.experimental.pallas import tpu_sc as plsc

NUM_EXPERTS = 8
TOP_K = 2
HIDDEN = 1024
INTER = 2048
TOKENS = 2048

RCHUNK = 256
NRC = TOKENS // RCHUNK
TILE_M = 256
NT = (TOKENS * TOP_K) // TILE_M + NUM_EXPERTS
M_PAD = NT * TILE_M

NC, NS, LANES = 2, 16, 16
NW = NC * NS


def _router_body(logits_ref, i0_ref, i1_ref, r0_ref, r1_ref, w0_ref, w1_ref,
                 cnt_ref, carry_ref):
    tc = pl.program_id(0)
    logits = logits_ref[...].astype(jnp.float32)
    m = jnp.max(logits, axis=1, keepdims=True)
    p = jnp.exp(logits - m)
    p = p / jnp.sum(p, axis=1, keepdims=True)
    eiota = lax.broadcasted_iota(jnp.int32, p.shape, 1)
    w0 = jnp.max(p, axis=1)
    i0 = jnp.min(jnp.where(p == w0[:, None], eiota, NUM_EXPERTS), axis=1)
    p1 = jnp.where(eiota == i0[:, None], -1.0, p)
    w1 = jnp.max(p1, axis=1)
    i1 = jnp.min(jnp.where(p1 == w1[:, None], eiota, NUM_EXPERTS), axis=1)
    denom = w0 + w1

    lane = lax.broadcasted_iota(jnp.int32, (RCHUNK, 128), 1)
    oh0 = (lane == i0[:, None]).astype(jnp.float32)
    oh1 = (lane == i1[:, None]).astype(jnp.float32)
    oh = (oh0 + oh1).astype(jnp.bfloat16)
    r = lax.broadcasted_iota(jnp.int32, (RCHUNK, RCHUNK), 0)
    c = lax.broadcasted_iota(jnp.int32, (RCHUNK, RCHUNK), 1)
    tril = (r > c).astype(jnp.bfloat16)
    cexcl = lax.dot_general(tril, oh, (((1,), (0,)), ((), ())),
                            preferred_element_type=jnp.float32)

    @pl.when(tc == 0)
    def _():
        carry_ref[...] = jnp.zeros_like(carry_ref)

    carry = carry_ref[...]
    cexcl = cexcl + carry
    r0 = jnp.sum(cexcl * oh0, axis=1)
    r1 = jnp.sum(cexcl * oh1, axis=1)
    new_carry = carry + jnp.sum(oh0 + oh1, axis=0, keepdims=True)
    carry_ref[...] = new_carry

    i0_ref[...] = i0.reshape(1, 1, RCHUNK)
    i1_ref[...] = i1.reshape(1, 1, RCHUNK)
    r0_ref[...] = r0.astype(jnp.int32).reshape(1, 1, RCHUNK)
    r1_ref[...] = r1.astype(jnp.int32).reshape(1, 1, RCHUNK)
    w0_ref[...] = (w0 / denom).reshape(1, 1, RCHUNK)
    w1_ref[...] = (w1 / denom).reshape(1, 1, RCHUNK)
    cnt_ref[...] = new_carry.astype(jnp.int32)


def _router(router_logits):
    outs = pl.pallas_call(
        _router_body,
        grid=(NRC,),
        in_specs=[pl.BlockSpec((RCHUNK, NUM_EXPERTS), lambda tc: (tc, 0))],
        out_specs=[
            pl.BlockSpec((1, 1, RCHUNK), lambda tc: (tc, 0, 0)),
            pl.BlockSpec((1, 1, RCHUNK), lambda tc: (tc, 0, 0)),
            pl.BlockSpec((1, 1, RCHUNK), lambda tc: (tc, 0, 0)),
            pl.BlockSpec((1, 1, RCHUNK), lambda tc: (tc, 0, 0)),
            pl.BlockSpec((1, 1, RCHUNK), lambda tc: (tc, 0, 0)),
            pl.BlockSpec((1, 1, RCHUNK), lambda tc: (tc, 0, 0)),
            pl.BlockSpec((1, 128), lambda tc: (0, 0)),
        ],
        out_shape=[
            jax.ShapeDtypeStruct((NRC, 1, RCHUNK), jnp.int32),
            jax.ShapeDtypeStruct((NRC, 1, RCHUNK), jnp.int32),
            jax.ShapeDtypeStruct((NRC, 1, RCHUNK), jnp.int32),
            jax.ShapeDtypeStruct((NRC, 1, RCHUNK), jnp.int32),
            jax.ShapeDtypeStruct((NRC, 1, RCHUNK), jnp.float32),
            jax.ShapeDtypeStruct((NRC, 1, RCHUNK), jnp.float32),
            jax.ShapeDtypeStruct((1, 128), jnp.int32),
        ],
        scratch_shapes=[pltpu.VMEM((1, 128), jnp.float32)],
        compiler_params=pltpu.CompilerParams(
            dimension_semantics=("arbitrary",)),
    )(router_logits)
    return outs


@functools.lru_cache(maxsize=1)
def _sc_mesh():
    return plsc.VectorSubcoreMesh(core_axis_name="c", subcore_axis_name="s",
                                  num_cores=NC, num_subcores=NS)


def _meta_body(cnt_hbm, i0_hbm, i1_hbm, r0_hbm, r1_hbm, w0_hbm, w1_hbm,
               src_hbm, wsort_hbm, pos0_hbm, pos1_hbm, te_hbm,
               cnt_v, starts_v, pc_v, i0_v, i1_v, r0_v, r1_v, w0_v, w1_v,
               src_v, wsort_v, pos0_v, pos1_v, te_v):
    cid = lax.axis_index("c")
    sid = lax.axis_index("s")

    @pl.when(jnp.logical_and(cid == 0, sid == 0))
    def _():
        pltpu.sync_copy(cnt_hbm.at[pl.ds(0, 16)], cnt_v)
        pltpu.sync_copy(i0_hbm, i0_v)
        pltpu.sync_copy(i1_hbm, i1_v)
        pltpu.sync_copy(r0_hbm, r0_v)
        pltpu.sync_copy(r1_hbm, r1_v)
        pltpu.sync_copy(w0_hbm, w0_v)
        pltpu.sync_copy(w1_hbm, w1_v)

        lane = lax.broadcasted_iota(jnp.int32, (LANES,), 0)
        c = cnt_v[...]
        pc = ((c + (TILE_M - 1)) >> 8) << 8
        pc = jnp.where(lane < NUM_EXPERTS, pc, 0)
        pc_v[...] = pc
        starts = jnp.zeros((LANES,), jnp.int32)
        for k in range(1, NUM_EXPERTS):
            g = plsc.load_gather(pc_v, [jnp.maximum(lane - k, 0)])
            starts = starts + jnp.where(lane >= k, g, 0)
        starts_v[...] = starts

        zeros16 = jnp.zeros((LANES,), jnp.int32)
        m0 = lane * TILE_M
        m1 = (lane + LANES) * TILE_M
        te0 = zeros16
        te1 = zeros16
        for e in range(1, NUM_EXPERTS):
            st_b = plsc.load_gather(starts_v,
                                    [jnp.full((LANES,), e, jnp.int32)])
            te0 = te0 + (m0 >= st_b).astype(jnp.int32)
            te1 = te1 + (m1 >= st_b).astype(jnp.int32)
        te_v[pl.ds(0, 16)] = jnp.clip(te0, 0, NUM_EXPERTS - 1)
        te_v[pl.ds(16, 16)] = jnp.clip(te1, 0, NUM_EXPERTS - 1)
        pltpu.sync_copy(te_v, te_hbm)

        def zero_body(i, _):
            src_v[pl.ds(i * LANES, LANES)] = zeros16
            wsort_v[pl.ds(i * LANES, LANES)] = jnp.zeros((LANES,), jnp.float32)
            return _

        lax.fori_loop(0, M_PAD // LANES, zero_body, None)

        def assign_body(i, _):
            base = i * LANES
            toks = base + lane
            e0 = i0_v[pl.ds(base, LANES)]
            p0 = plsc.load_gather(starts_v, [e0]) + r0_v[pl.ds(base, LANES)]
            pos0_v[pl.ds(base, LANES)] = p0
            plsc.store_scatter(src_v, [p0], toks)
            plsc.store_scatter(wsort_v, [p0], w0_v[pl.ds(base, LANES)])
            e1 = i1_v[pl.ds(base, LANES)]
            p1 = plsc.load_gather(starts_v, [e1]) + r1_v[pl.ds(base, LANES)]
            pos1_v[pl.ds(base, LANES)] = p1
            plsc.store_scatter(src_v, [p1], toks)
            plsc.store_scatter(wsort_v, [p1], w1_v[pl.ds(base, LANES)])
            return _

        lax.fori_loop(0, TOKENS // LANES, assign_body, None)

        pltpu.sync_copy(src_v, src_hbm)
        pltpu.sync_copy(wsort_v, wsort_hbm)
        pltpu.sync_copy(pos0_v, pos0_hbm)
        pltpu.sync_copy(pos1_v, pos1_hbm)


def _metadata(cnt, i0, i1, r0, r1, w0, w1):
    f = pl.kernel(
        _meta_body,
        out_type=[
            jax.ShapeDtypeStruct((M_PAD,), jnp.int32),
            jax.ShapeDtypeStruct((M_PAD,), jnp.float32),
            jax.ShapeDtypeStruct((TOKENS,), jnp.int32),
            jax.ShapeDtypeStruct((TOKENS,), jnp.int32),
            jax.ShapeDtypeStruct((NW,), jnp.int32),
        ],
        mesh=_sc_mesh(),
        compiler_params=pltpu.CompilerParams(needs_layout_passes=False),
        scratch_types=[
            pltpu.VMEM((LANES,), jnp.int32),
            pltpu.VMEM((LANES,), jnp.int32),
            pltpu.VMEM((LANES,), jnp.int32),
            pltpu.VMEM((TOKENS,), jnp.int32),
            pltpu.VMEM((TOKENS,), jnp.int32),
            pltpu.VMEM((TOKENS,), jnp.int32),
            pltpu.VMEM((TOKENS,), jnp.int32),
            pltpu.VMEM((TOKENS,), jnp.float32),
            pltpu.VMEM((TOKENS,), jnp.float32),
            pltpu.VMEM((M_PAD,), jnp.int32),
            pltpu.VMEM((M_PAD,), jnp.float32),
            pltpu.VMEM((TOKENS,), jnp.int32),
            pltpu.VMEM((TOKENS,), jnp.int32),
            pltpu.VMEM((NW,), jnp.int32),
        ],
    )
    return f(cnt, i0, i1, r0, r1, w0, w1)


G_ROWS = M_PAD // NW
G_CHUNK = 96


def _gather_body(src_hbm, x_hbm, xs_hbm, idx_v, buf_v, sem):
    wid = lax.axis_index("s") * NC + lax.axis_index("c")
    base = wid * G_ROWS

    def chunk(j, _):
        b = base + j * G_CHUNK
        pltpu.sync_copy(src_hbm.at[pl.ds(b, G_CHUNK)], idx_v)
        pltpu.async_copy(x_hbm.at[idx_v], buf_v, sem).wait()
        pltpu.sync_copy(buf_v, xs_hbm.at[pl.ds(b, G_CHUNK)])
        return _

    lax.fori_loop(0, G_ROWS // G_CHUNK, chunk, None)


def _gather(src, hidden_states):
    f = pl.kernel(
        _gather_body,
        out_type=[jax.ShapeDtypeStruct((M_PAD, HIDDEN), jnp.float32)],
        mesh=_sc_mesh(),
        compiler_params=pltpu.CompilerParams(needs_layout_passes=False),
        scratch_types=[
            pltpu.VMEM((G_CHUNK,), jnp.int32),
            pltpu.VMEM((G_CHUNK, HIDDEN), jnp.float32),
            pltpu.SemaphoreType.DMA,
        ],
    )
    return f(src, hidden_states)[0]


def _mm_body(te_ref, src_ref, x_ref, w13_ref, w2_ref, ws_ref, o_ref):
    s = src_ref[0, 0, :]
    tok = lax.broadcasted_iota(jnp.int32, (TILE_M, TOKENS), 1)
    perm = (tok == s[:, None]).astype(jnp.bfloat16)
    xt = lax.dot_general(perm, x_ref[...], (((1,), (0,)), ((), ())),
                         preferred_element_type=jnp.float32)
    x = xt.astype(jnp.bfloat16)
    w13 = w13_ref[0].astype(jnp.bfloat16)
    w2 = w2_ref[0].astype(jnp.bfloat16)
    h = lax.dot_general(x, w13, (((1,), (1,)), ((), ())),
                        preferred_element_type=jnp.float32)
    gate = h[:, :INTER]
    up = h[:, INTER:]
    act = (gate * jax.nn.sigmoid(gate) * up).astype(jnp.bfloat16)
    y = lax.dot_general(act, w2, (((1,), (1,)), ((), ())),
                        preferred_element_type=jnp.float32)
    o_ref[...] = y * ws_ref[0, 0, :][:, None]


def _grouped_matmul(te, src, x_bf, w13_bf, w2_bf, wsort):
    grid_spec = pltpu.PrefetchScalarGridSpec(
        num_scalar_prefetch=1,
        grid=(NT,),
        in_specs=[
            pl.BlockSpec((1, 1, TILE_M), lambda j, te: (j, 0, 0)),
            pl.BlockSpec((TOKENS, HIDDEN), lambda j, te: (0, 0)),
            pl.BlockSpec((1, 2 * INTER, HIDDEN), lambda j, te: (te[j], 0, 0)),
            pl.BlockSpec((1, HIDDEN, INTER), lambda j, te: (te[j], 0, 0)),
            pl.BlockSpec((1, 1, TILE_M), lambda j, te: (j, 0, 0)),
        ],
        out_specs=pl.BlockSpec((TILE_M, HIDDEN), lambda j, te: (j, 0)),
    )
    return pl.pallas_call(
        _mm_body,
        grid_spec=grid_spec,
        out_shape=jax.ShapeDtypeStruct((M_PAD, HIDDEN), jnp.float32),
        compiler_params=pltpu.CompilerParams(
            dimension_semantics=("arbitrary",),
            vmem_limit_bytes=115 * 1024 * 1024),
    )(te, src.reshape(NT, 1, TILE_M), x_bf, w13_bf, w2_bf,
      wsort.reshape(NT, 1, TILE_M))


C_TOKS = TOKENS // NW
C_CHUNK = 32


def _combine_body(pos0_hbm, pos1_hbm, ys_hbm, out_hbm,
                  p0_v, p1_v, buf0_v, buf1_v, sem0, sem1):
    wid = lax.axis_index("s") * NC + lax.axis_index("c")
    base = wid * C_TOKS

    def chunk(j, _):
        b = base + j * C_CHUNK
        pltpu.sync_copy(pos0_hbm.at[pl.ds(b, C_CHUNK)], p0_v)
        pltpu.sync_copy(pos1_hbm.at[pl.ds(b, C_CHUNK)], p1_v)
        cp0 = pltpu.async_copy(ys_hbm.at[p0_v], buf0_v, sem0)
        cp1 = pltpu.async_copy(ys_hbm.at[p1_v], buf1_v, sem1)
        cp0.wait()
        cp1.wait()

        def add_body(it, _):
            row = it // (HIDDEN // LANES)
            off = (it % (HIDDEN // LANES)) * LANES
            buf0_v[row, pl.ds(off, LANES)] = (buf0_v[row, pl.ds(off, LANES)]
                                              + buf1_v[row, pl.ds(off, LANES)])
            return _

        lax.fori_loop(0, (C_CHUNK * HIDDEN) // LANES, add_body, None)
        pltpu.sync_copy(buf0_v, out_hbm.at[pl.ds(b, C_CHUNK)])
        return _

    lax.fori_loop(0, C_TOKS // C_CHUNK, chunk, None)


def _combine(pos0, pos1, ysort):
    f = pl.kernel(
        _combine_body,
        out_type=[jax.ShapeDtypeStruct((TOKENS, HIDDEN), jnp.float32)],
        mesh=_sc_mesh(),
        compiler_params=pltpu.CompilerParams(needs_layout_passes=False),
        scratch_types=[
            pltpu.VMEM((C_CHUNK,), jnp.int32),
            pltpu.VMEM((C_CHUNK,), jnp.int32),
            pltpu.VMEM((C_CHUNK, HIDDEN), jnp.float32),
            pltpu.VMEM((C_CHUNK, HIDDEN), jnp.float32),
            pltpu.SemaphoreType.DMA,
            pltpu.SemaphoreType.DMA,
        ],
    )
    return f(pos0, pos1, ysort)[0]


@jax.jit
def kernel(hidden_states, router_logits, w13_weight, w2_weight):
    i0, i1, r0, r1, w0, w1, cnt = _router(router_logits)
    src, wsort, pos0, pos1, te = _metadata(
        cnt.reshape(128),
        i0.reshape(TOKENS), i1.reshape(TOKENS),
        r0.reshape(TOKENS), r1.reshape(TOKENS),
        w0.reshape(TOKENS), w1.reshape(TOKENS))
    x_bf = hidden_states.astype(jnp.bfloat16)
    ysort = _grouped_matmul(te, src, x_bf, w13_weight, w2_weight, wsort)
    return _combine(pos0, pos1, ysort)
